# baseline (speedup 1.0000x reference)
import functools

import jax
import jax.numpy as jnp
from jax import lax
from jax.experimental import pallas as pl
from jax.experimental.pallas import tpu as pltpu
from jax.experimental.pallas import tpu_sc as plsc

NUM_EMBED = 1000000
EMBED_DIM = 64
OUTPUT_DIM = 128
BATCH = 16384
FIELDS = 26
ROWS = BATCH * FIELDS
HALF2 = 500224

NC = 2
NS = 16
NW = NC * NS
RPW = ROWS // NW
CH = 128
NCH = RPW // CH
GRP = CH // 16

CVT_BO = 512


def _cvt_body(a_ref, b_ref, o_ref):
    o_ref[:, :EMBED_DIM] = a_ref[...].T
    o_ref[:, EMBED_DIM:] = b_ref[...].T


def _convert(tab_t):
    nblk = HALF2 // CVT_BO
    return pl.pallas_call(
        _cvt_body,
        grid=(nblk,),
        in_specs=[
            pl.BlockSpec((EMBED_DIM, CVT_BO), lambda i: (0, i)),
            pl.BlockSpec((EMBED_DIM, CVT_BO), lambda i: (0, i + nblk)),
        ],
        out_specs=pl.BlockSpec((CVT_BO, 2 * EMBED_DIM), lambda i: (i, 0)),
        out_shape=jax.ShapeDtypeStruct((HALF2, 2 * EMBED_DIM), jnp.float32),
    )(tab_t, tab_t)


def _gather_body(idx_hbm, t2_hbm, h2_hbm, idx_v, rows_v, gsem, ssem):
    wid = lax.axis_index("s") * NC + lax.axis_index("c")
    hbase = wid * (RPW // 2)
    pltpu.sync_copy(idx_hbm.at[wid], idx_v)

    def fire_burst(c, slot):
        lanes = lax.iota(jnp.int32, 16)
        for g in range(GRP):
            vec = idx_v[pl.ds(c * CH + g * 16, 16)]
            for j in range(16):
                r = jnp.sum(vec * (lanes == j).astype(jnp.int32))
                p = jnp.where(r < HALF2, r, r - HALF2)
                off = jnp.where(r < HALF2, 0, EMBED_DIM)
                jj = g * 16 + j
                pltpu.make_async_copy(
                    t2_hbm.at[pl.ds(p, 1), pl.ds(off, EMBED_DIM)],
                    rows_v.at[slot, pl.ds(jj // 2, 1),
                              pl.ds((jj % 2) * EMBED_DIM, EMBED_DIM)],
                    gsem.at[slot],
                ).start()

    def wait_burst(slot):
        pltpu.make_async_copy(
            t2_hbm.at[pl.ds(0, CH // 2)],
            rows_v.at[slot],
            gsem.at[slot],
        ).wait()

    def fire_store(c, slot):
        pltpu.make_async_copy(
            rows_v.at[slot],
            h2_hbm.at[pl.ds(hbase + c * (CH // 2), CH // 2)],
            ssem.at[slot],
        ).start()

    def wait_store(slot):
        pltpu.make_async_copy(
            rows_v.at[slot],
            h2_hbm.at[pl.ds(hbase, CH // 2)],
            ssem.at[slot],
        ).wait()

    fire_burst(0, 0)

    def step(c, carry):
        slot = c % 2

        @pl.when(c + 1 < NCH)
        def _():
            @pl.when(c >= 1)
            def _():
                wait_store(1 - slot)

            fire_burst(c + 1, 1 - slot)

        wait_burst(slot)
        fire_store(c, slot)
        return carry

    lax.fori_loop(0, NCH, step, 0)
    wait_store(0)
    wait_store(1)


@functools.cache
def _make_gather():
    return pl.kernel(
        _gather_body,
        mesh=plsc.VectorSubcoreMesh(core_axis_name="c", subcore_axis_name="s"),
        out_type=jax.ShapeDtypeStruct((ROWS // 2, 2 * EMBED_DIM), jnp.float32),
        compiler_params=pltpu.CompilerParams(needs_layout_passes=False),
        scratch_types=[
            pltpu.VMEM((RPW,), jnp.int32),
            pltpu.VMEM((2, CH // 2, 2 * EMBED_DIM), jnp.float32),
            pltpu.SemaphoreType.DMA((2,)),
            pltpu.SemaphoreType.DMA((2,)),
        ],
    )


MM_BLK = 1024


def _mm_body(h_ref, wt_ref, b_ref, o_ref):
    o_ref[...] = (
        jnp.dot(h_ref[...], wt_ref[...], preferred_element_type=jnp.float32)
        + b_ref[...]
    )


def _matmul(h2, w2, b2):
    return pl.pallas_call(
        _mm_body,
        grid=(ROWS // 2 // MM_BLK,),
        in_specs=[
            pl.BlockSpec((MM_BLK, 2 * EMBED_DIM), lambda i: (i, 0)),
            pl.BlockSpec((2 * EMBED_DIM, 2 * OUTPUT_DIM), lambda i: (0, 0)),
            pl.BlockSpec((1, 2 * OUTPUT_DIM), lambda i: (0, 0)),
        ],
        out_specs=pl.BlockSpec((MM_BLK, 2 * OUTPUT_DIM), lambda i: (i, 0)),
        out_shape=jax.ShapeDtypeStruct((ROWS // 2, 2 * OUTPUT_DIM), jnp.float32),
    )(h2, w2, b2)


def kernel(x, table, W, b):
    idx = x.T.reshape(NW, RPW).astype(jnp.int32)
    t2 = _convert(table.T)
    h2 = _make_gather()(idx, t2)
    wt = W.T
    z = jnp.zeros((EMBED_DIM, OUTPUT_DIM), jnp.float32)
    w2 = jnp.concatenate(
        [jnp.concatenate([wt, z], axis=1), jnp.concatenate([z, wt], axis=1)],
        axis=0,
    )
    b2 = jnp.concatenate([b, b]).reshape(1, 2 * OUTPUT_DIM)
    out2 = _matmul(h2, w2, b2)
    return out2.reshape(FIELDS, BATCH, OUTPUT_DIM).transpose(1, 0, 2)


# speedup vs baseline: 1.7372x; 1.7372x over previous
import functools

import jax
import jax.numpy as jnp
from jax import lax
from jax.experimental import pallas as pl
from jax.experimental.pallas import tpu as pltpu
from jax.experimental.pallas import tpu_sc as plsc

NUM_EMBED = 1000000
EMBED_DIM = 64
OUTPUT_DIM = 128
BATCH = 16384
FIELDS = 26
ROWS = BATCH * FIELDS
NC = 2
NS = 16
NW = NC * NS
K = 4
RCHUNK = ROWS // K
RPW = RCHUNK // NW
CH = 128
NCH = RPW // CH
GRP = CH // 16

def _gather_body(idx_hbm, tab_hbm, h2_hbm, idx_v, rows_v, gsem, ssem):
    wid = lax.axis_index("s") * NC + lax.axis_index("c")
    hbase = wid * RPW
    pltpu.sync_copy(idx_hbm.at[wid], idx_v)

    def fire_burst(c, slot):
        lanes = lax.iota(jnp.int32, 16)
        for g in range(GRP):
            vec = idx_v[pl.ds(c * CH + g * 16, 16)]
            for j in range(16):
                r = jnp.sum(vec * (lanes == j).astype(jnp.int32))
                pltpu.make_async_copy(
                    tab_hbm.at[pl.ds(r, 1)],
                    rows_v.at[slot, pl.ds(g * 16 + j, 1)],
                    gsem.at[slot],
                ).start()

    def wait_burst(slot):
        pltpu.make_async_copy(
            tab_hbm.at[pl.ds(0, CH)],
            rows_v.at[slot],
            gsem.at[slot],
        ).wait()

    def fire_store(c, slot):
        pltpu.make_async_copy(
            rows_v.at[slot],
            h2_hbm.at[pl.ds(hbase + c * CH, CH)],
            ssem.at[slot],
        ).start()

    def wait_store(slot):
        pltpu.make_async_copy(
            rows_v.at[slot],
            h2_hbm.at[pl.ds(hbase, CH)],
            ssem.at[slot],
        ).wait()

    fire_burst(0, 0)

    def step(c, carry):
        slot = c % 2

        @pl.when(c + 1 < NCH)
        def _():
            @pl.when(c >= 1)
            def _():
                wait_store(1 - slot)

            fire_burst(c + 1, 1 - slot)

        wait_burst(slot)
        fire_store(c, slot)
        return carry

    lax.fori_loop(0, NCH, step, 0)
    wait_store(0)
    wait_store(1)


@functools.cache
def _make_gather():
    return pl.kernel(
        _gather_body,
        mesh=plsc.VectorSubcoreMesh(core_axis_name="c", subcore_axis_name="s"),
        out_type=jax.ShapeDtypeStruct((RCHUNK, EMBED_DIM), jnp.float32),
        compiler_params=pltpu.CompilerParams(needs_layout_passes=False),
        scratch_types=[
            pltpu.VMEM((RPW,), jnp.int32),
            pltpu.VMEM((2, CH, EMBED_DIM), jnp.float32),
            pltpu.SemaphoreType.DMA((2,)),
            pltpu.SemaphoreType.DMA((2,)),
        ],
    )


MM_BLK = 2048
MM_GRID = RCHUNK // MM_BLK


def _mm_body0(h_ref, wt_ref, b_ref, o_ref):
    o_ref[...] = (
        jnp.dot(h_ref[...], wt_ref[...], preferred_element_type=jnp.float32)
        + b_ref[...]
    )


def _mm_bodyk(h_ref, wt_ref, b_ref, carry_ref, o_ref):
    del carry_ref
    o_ref[...] = (
        jnp.dot(h_ref[...], wt_ref[...], preferred_element_type=jnp.float32)
        + b_ref[...]
    )


def _matmul_chunk(h2k, w2, b2, k, out2_prev):
    if k == 0:
        return pl.pallas_call(
            _mm_body0,
            grid=(MM_GRID,),
            in_specs=[
                pl.BlockSpec((MM_BLK, EMBED_DIM), lambda i: (i, 0)),
                pl.BlockSpec((EMBED_DIM, OUTPUT_DIM), lambda i: (0, 0)),
                pl.BlockSpec((1, OUTPUT_DIM), lambda i: (0, 0)),
            ],
            out_specs=pl.BlockSpec((MM_BLK, OUTPUT_DIM), lambda i: (i, 0)),
            out_shape=jax.ShapeDtypeStruct((ROWS, OUTPUT_DIM), jnp.float32),
        )(h2k, w2, b2)
    return pl.pallas_call(
        _mm_bodyk,
        grid=(MM_GRID,),
        in_specs=[
            pl.BlockSpec((MM_BLK, EMBED_DIM), lambda i: (i, 0)),
            pl.BlockSpec((EMBED_DIM, OUTPUT_DIM), lambda i: (0, 0)),
            pl.BlockSpec((1, OUTPUT_DIM), lambda i: (0, 0)),
            pl.BlockSpec(memory_space=pl.ANY),
        ],
        out_specs=pl.BlockSpec((MM_BLK, OUTPUT_DIM),
                               lambda i, k=k: (k * MM_GRID + i, 0)),
        out_shape=jax.ShapeDtypeStruct((ROWS, OUTPUT_DIM), jnp.float32),
        input_output_aliases={3: 0},
    )(h2k, w2, b2, out2_prev)


def kernel(x, table, W, b):
    idx = x.T.reshape(ROWS).astype(jnp.int32)
    wt = W.T
    b2 = b.reshape(1, OUTPUT_DIM)
    gather = _make_gather()
    out2 = None
    for k in range(K):
        idx_k = lax.slice(idx, (k * RCHUNK,), ((k + 1) * RCHUNK,)).reshape(
            NW, RPW)
        h2k = gather(idx_k, table)
        out2 = _matmul_chunk(h2k, wt, b2, k, out2)
    return out2.reshape(FIELDS, BATCH, OUTPUT_DIM).transpose(1, 0, 2)
